# Initial kernel scaffold; baseline (speedup 1.0000x reference)
import jax
import jax.numpy as jnp
from jax.experimental import pallas as pl


def kernel(x, embedding):
    raise NotImplementedError("write your pallas kernel here")


# speedup vs baseline: 1.7993x; 1.7993x over previous
import jax
import jax.numpy as jnp
from jax.experimental import pallas as pl

BATCH = 4
NUM_POSITIONS = 8192
FEATURE_DIM = 768
POS_BLOCK = 512


def _body(x_ref, e_ref, o_ref):
    o_ref[...] = x_ref[...] + e_ref[...][None, :, :]


def kernel(x, embedding):
    grid = (NUM_POSITIONS // POS_BLOCK,)
    return pl.pallas_call(
        _body,
        grid=grid,
        in_specs=[
            pl.BlockSpec((BATCH, POS_BLOCK, FEATURE_DIM), lambda i: (0, i, 0)),
            pl.BlockSpec((POS_BLOCK, FEATURE_DIM), lambda i: (i, 0)),
        ],
        out_specs=pl.BlockSpec((BATCH, POS_BLOCK, FEATURE_DIM), lambda i: (0, i, 0)),
        out_shape=jax.ShapeDtypeStruct((BATCH, NUM_POSITIONS, FEATURE_DIM), jnp.float32),
    )(x, embedding)


# speedup vs baseline: 1.8101x; 1.0060x over previous
import jax
import jax.numpy as jnp
from jax.experimental import pallas as pl

BATCH = 4
NUM_POSITIONS = 8192
FEATURE_DIM = 768
POS_BLOCK = 1024


def _body(x_ref, e_ref, o_ref):
    o_ref[...] = x_ref[...] + e_ref[...][None, :, :]


def kernel(x, embedding):
    grid = (NUM_POSITIONS // POS_BLOCK,)
    return pl.pallas_call(
        _body,
        grid=grid,
        in_specs=[
            pl.BlockSpec((BATCH, POS_BLOCK, FEATURE_DIM), lambda i: (0, i, 0)),
            pl.BlockSpec((POS_BLOCK, FEATURE_DIM), lambda i: (i, 0)),
        ],
        out_specs=pl.BlockSpec((BATCH, POS_BLOCK, FEATURE_DIM), lambda i: (0, i, 0)),
        out_shape=jax.ShapeDtypeStruct((BATCH, NUM_POSITIONS, FEATURE_DIM), jnp.float32),
    )(x, embedding)
